# baseline (speedup 1.0000x reference)
import functools

import jax
import jax.numpy as jnp
import numpy as np
from jax import lax
from jax.experimental import pallas as pl
from jax.experimental.pallas import tpu as pltpu
from jax.experimental.pallas import tpu_sc as plsc

NPIX = 196608
NFREQS = 128
NPTS = 65536

NW = 32
PTS_PER_W = NPTS // NW
PTS_PER_SUB = 64
GPS = 2
ROWS_PER_G = 128
SUBS = PTS_PER_W // PTS_PER_SUB
NIDX = SUBS * GPS
LANES = 16
SLICES = NFREQS // LANES
BLOCKS = NFREQS // 32

_PERM = np.array(
    [(m // 32) * 32 + ((m % 32) // 2 if m % 2 == 0 else 16 + (m % 32) // 2)
     for m in range(NFREQS)],
    dtype=np.int32,
)


def _sc_gather(table, idx3, wgt3):
    mesh = plsc.VectorSubcoreMesh(core_axis_name="c", subcore_axis_name="s")

    @functools.partial(
        pl.kernel,
        out_type=jax.ShapeDtypeStruct((NPTS, NFREQS), jnp.float32),
        mesh=mesh,
        compiler_params=pltpu.CompilerParams(use_tc_tiling_on_sc=False),
        scratch_types=[
            pltpu.VMEM((NIDX, ROWS_PER_G), jnp.int32),
            pltpu.VMEM((NIDX, ROWS_PER_G), jnp.float32),
            pltpu.VMEM((2, PTS_PER_SUB * 4, NFREQS // 2), jnp.uint32),
            pltpu.VMEM((2, PTS_PER_SUB, NFREQS), jnp.float32),
            pltpu.SemaphoreType.DMA,
            pltpu.SemaphoreType.DMA,
            pltpu.SemaphoreType.DMA,
            pltpu.SemaphoreType.DMA,
        ],
    )
    def k(table_hbm, idx_hbm, wgt_hbm, out_hbm, idx_v, wgt_v, buf, outb,
          gsem0, gsem1, osem0, osem1):
        gsems = (gsem0, gsem1)
        osems = (osem0, osem1)
        wid = lax.axis_index("s") * 2 + lax.axis_index("c")
        base = wid * PTS_PER_W
        pltpu.sync_copy(idx_hbm.at[wid], idx_v)
        pltpu.sync_copy(wgt_hbm.at[wid], wgt_v)

        def start_gather(g, u):
            for h in range(GPS):
                pltpu.async_copy(
                    table_hbm.at[idx_v.at[GPS * g + h]],
                    buf.at[u, pl.ds(h * ROWS_PER_G, ROWS_PER_G)],
                    gsems[u],
                )

        def wait_gather(g, u):
            for h in range(GPS):
                pltpu.make_async_copy(
                    table_hbm.at[idx_v.at[GPS * g + h]],
                    buf.at[u, pl.ds(h * ROWS_PER_G, ROWS_PER_G)],
                    gsems[u],
                ).wait()

        for u in range(2):
            start_gather(u, u)

        def pair_body(tq, carry):
            for u in range(2):
                g = 2 * tq + u
                wait_gather(g, u)

                @pl.when(tq >= 1)
                def _():
                    pltpu.make_async_copy(
                        outb.at[u],
                        out_hbm.at[pl.ds(base + (g - 2) * PTS_PER_SUB,
                                         PTS_PER_SUB)],
                        osems[u],
                    ).wait()

                @plsc.parallel_loop(0, PTS_PER_SUB // 4, unroll=2)
                def _(q, u=u):
                    wv = wgt_v[GPS * g + q // 8, pl.ds((q % 8) * LANES, LANES)]
                    himask = jnp.full((LANES,), 0xFFFF0000, dtype=jnp.uint32)
                    for pp in range(4):
                        p = q * 4 + pp
                        w = [
                            jnp.full((LANES,), wv[4 * pp + j],
                                     dtype=jnp.float32)
                            for j in range(4)
                        ]
                        for blk in range(BLOCKS):
                            sl = pl.ds(blk * LANES, LANES)
                            acc_e = None
                            acc_o = None
                            for j in range(4):
                                uw = buf[u, 4 * p + j, sl]
                                e = lax.bitcast_convert_type(uw << 16,
                                                             jnp.float32)
                                o = lax.bitcast_convert_type(uw & himask,
                                                             jnp.float32)
                                te = w[j] * e
                                to = w[j] * o
                                acc_e = te if acc_e is None else acc_e + te
                                acc_o = to if acc_o is None else acc_o + to
                            outb[u, p, pl.ds(blk * 32, LANES)] = acc_e
                            outb[u, p, pl.ds(blk * 32 + LANES, LANES)] = acc_o

                @pl.when(g + 2 < SUBS)
                def _():
                    start_gather(g + 2, u)

                pltpu.async_copy(
                    outb.at[u],
                    out_hbm.at[pl.ds(base + g * PTS_PER_SUB, PTS_PER_SUB)],
                    osems[u],
                )
            return carry

        lax.fori_loop(0, SUBS // 2, pair_body, 0, unroll=False)

        for u in range(2):
            pltpu.make_async_copy(
                outb.at[u],
                out_hbm.at[pl.ds(base + (SUBS - 2 + u) * PTS_PER_SUB,
                                 PTS_PER_SUB)],
                osems[u],
            ).wait()

    return k(table, idx3, wgt3)


def kernel(params, inds, wgts, freqs):
    fmap = params.reshape(NFREQS, NPIX)
    tb = fmap[jnp.asarray(_PERM)].T.astype(jnp.bfloat16)
    table = lax.bitcast_convert_type(
        tb.reshape(NPIX, NFREQS // 2, 2), jnp.uint32)
    idx3 = inds.astype(jnp.int32).reshape(NW, NIDX, ROWS_PER_G)
    wgt3 = wgts.astype(jnp.float32).reshape(NW, NIDX, ROWS_PER_G)
    out = _sc_gather(table, idx3, wgt3)
    return out.T.reshape(1, 1, 1, NFREQS, NPTS)


# speedup vs baseline: 2.9646x; 2.9646x over previous
import functools

import jax
import jax.numpy as jnp
from jax import lax
from jax.experimental import pallas as pl
from jax.experimental.pallas import tpu as pltpu
from jax.experimental.pallas import tpu_sc as plsc

NPIX = 196608
NFREQS = 128
NPTS = 65536

NW = 32
PTS_PER_W = NPTS // NW
PTS_PER_SUB = 64
GPS = 2
ROWS_PER_G = 128
SUBS = PTS_PER_W // PTS_PER_SUB
NIDX = SUBS * GPS
LANES = 16
SLICES = NFREQS // LANES


def _sc_gather(table, idx3, wgt3):
    mesh = plsc.VectorSubcoreMesh(core_axis_name="c", subcore_axis_name="s")

    @functools.partial(
        pl.kernel,
        out_type=jax.ShapeDtypeStruct((NPTS, NFREQS), jnp.float32),
        mesh=mesh,
        scratch_types=[
            pltpu.VMEM((NIDX, ROWS_PER_G), jnp.int32),
            pltpu.VMEM((NIDX, ROWS_PER_G), jnp.float32),
            pltpu.VMEM((2, PTS_PER_SUB * 4, NFREQS), jnp.float32),
            pltpu.VMEM((2, PTS_PER_SUB, NFREQS), jnp.float32),
            pltpu.SemaphoreType.DMA,
            pltpu.SemaphoreType.DMA,
            pltpu.SemaphoreType.DMA,
            pltpu.SemaphoreType.DMA,
        ],
    )
    def k(table_hbm, idx_hbm, wgt_hbm, out_hbm, idx_v, wgt_v, buf, outb,
          gsem0, gsem1, osem0, osem1):
        gsems = (gsem0, gsem1)
        osems = (osem0, osem1)
        wid = lax.axis_index("s") * 2 + lax.axis_index("c")
        base = wid * PTS_PER_W
        pltpu.sync_copy(idx_hbm.at[wid], idx_v)
        pltpu.sync_copy(wgt_hbm.at[wid], wgt_v)

        def start_gather(g, u):
            for h in range(GPS):
                pltpu.async_copy(
                    table_hbm.at[idx_v.at[GPS * g + h]],
                    buf.at[u, pl.ds(h * ROWS_PER_G, ROWS_PER_G)],
                    gsems[u],
                )

        def wait_gather(g, u):
            for h in range(GPS):
                pltpu.make_async_copy(
                    table_hbm.at[idx_v.at[GPS * g + h]],
                    buf.at[u, pl.ds(h * ROWS_PER_G, ROWS_PER_G)],
                    gsems[u],
                ).wait()

        for u in range(2):
            start_gather(u, u)

        def pair_body(tq, carry):
            for u in range(2):
                g = 2 * tq + u
                wait_gather(g, u)

                @pl.when(tq >= 1)
                def _():
                    pltpu.make_async_copy(
                        outb.at[u],
                        out_hbm.at[pl.ds(base + (g - 2) * PTS_PER_SUB,
                                         PTS_PER_SUB)],
                        osems[u],
                    ).wait()

                @plsc.parallel_loop(0, PTS_PER_SUB // 4, unroll=4)
                def _(q, u=u):
                    wv = wgt_v[GPS * g + q // 8, pl.ds((q % 8) * LANES, LANES)]
                    for pp in range(4):
                        p = q * 4 + pp
                        w = [
                            jnp.full((LANES,), wv[4 * pp + j],
                                     dtype=jnp.float32)
                            for j in range(4)
                        ]
                        for s in range(SLICES):
                            sl = pl.ds(s * LANES, LANES)
                            a = (w[0] * buf[u, 4 * p + 0, sl]
                                 + w[1] * buf[u, 4 * p + 1, sl])
                            b2 = (w[2] * buf[u, 4 * p + 2, sl]
                                  + w[3] * buf[u, 4 * p + 3, sl])
                            outb[u, p, sl] = a + b2

                @pl.when(g + 2 < SUBS)
                def _():
                    start_gather(g + 2, u)

                pltpu.async_copy(
                    outb.at[u],
                    out_hbm.at[pl.ds(base + g * PTS_PER_SUB, PTS_PER_SUB)],
                    osems[u],
                )
            return carry

        lax.fori_loop(0, SUBS // 2, pair_body, 0, unroll=False)

        for u in range(2):
            pltpu.make_async_copy(
                outb.at[u],
                out_hbm.at[pl.ds(base + (SUBS - 2 + u) * PTS_PER_SUB,
                                 PTS_PER_SUB)],
                osems[u],
            ).wait()

    return k(table, idx3, wgt3)


def kernel(params, inds, wgts, freqs):
    table = params.reshape(NFREQS, NPIX).T
    idx3 = inds.astype(jnp.int32).reshape(NW, NIDX, ROWS_PER_G)
    wgt3 = wgts.astype(jnp.float32).reshape(NW, NIDX, ROWS_PER_G)
    out = _sc_gather(table, idx3, wgt3)
    return out.T.reshape(1, 1, 1, NFREQS, NPTS)


# speedup vs baseline: 3.0340x; 1.0234x over previous
import functools

import jax
import jax.numpy as jnp
from jax import lax
from jax.experimental import pallas as pl
from jax.experimental.pallas import tpu as pltpu
from jax.experimental.pallas import tpu_sc as plsc

NPIX = 196608
NFREQS = 128
NPTS = 65536

NW = 32
PTS_PER_W = NPTS // NW
PTS_PER_SUB = 64
GPS = 2
ROWS_PER_G = 128
SUBS = PTS_PER_W // PTS_PER_SUB
NIDX = SUBS * GPS
LANES = 16
SLICES = NFREQS // LANES


def _sc_gather(table, idx3, wgt3):
    mesh = plsc.VectorSubcoreMesh(core_axis_name="c", subcore_axis_name="s")

    @functools.partial(
        pl.kernel,
        out_type=jax.ShapeDtypeStruct((NPTS, NFREQS), jnp.float32),
        mesh=mesh,
        scratch_types=[
            pltpu.VMEM((NIDX, ROWS_PER_G), jnp.int32),
            pltpu.VMEM((NIDX, ROWS_PER_G), jnp.float32),
            pltpu.VMEM((2, PTS_PER_SUB * 4, NFREQS), jnp.float32),
            pltpu.VMEM((2, PTS_PER_SUB, NFREQS), jnp.float32),
            pltpu.SemaphoreType.DMA,
            pltpu.SemaphoreType.DMA,
            pltpu.SemaphoreType.DMA,
            pltpu.SemaphoreType.DMA,
        ],
    )
    def k(table_hbm, idx_hbm, wgt_hbm, out_hbm, idx_v, wgt_v, buf, outb,
          gsem0, gsem1, osem0, osem1):
        gsems = (gsem0, gsem1)
        osems = (osem0, osem1)
        wid = lax.axis_index("s") * 2 + lax.axis_index("c")
        base = wid * PTS_PER_W
        pltpu.sync_copy(idx_hbm.at[wid], idx_v)
        pltpu.sync_copy(wgt_hbm.at[wid], wgt_v)

        def start_gather(g, u):
            for h in range(GPS):
                pltpu.async_copy(
                    table_hbm.at[idx_v.at[GPS * g + h]],
                    buf.at[u, pl.ds(h * ROWS_PER_G, ROWS_PER_G)],
                    gsems[u],
                )

        def wait_gather(g, u):
            for h in range(GPS):
                pltpu.make_async_copy(
                    table_hbm.at[idx_v.at[GPS * g + h]],
                    buf.at[u, pl.ds(h * ROWS_PER_G, ROWS_PER_G)],
                    gsems[u],
                ).wait()

        for u in range(2):
            start_gather(u, u)

        def pair_body(tq, carry):
            for u in range(2):
                g = 2 * tq + u
                wait_gather(g, u)

                @pl.when(tq >= 1)
                def _():
                    pltpu.make_async_copy(
                        outb.at[u],
                        out_hbm.at[pl.ds(base + (g - 2) * PTS_PER_SUB,
                                         PTS_PER_SUB)],
                        osems[u],
                    ).wait()

                @plsc.parallel_loop(0, PTS_PER_SUB // 4, unroll=2)
                def _(q, u=u):
                    wv = wgt_v[GPS * g + q // 8, pl.ds((q % 8) * LANES, LANES)]
                    for pp in range(4):
                        p = q * 4 + pp
                        w = [
                            jnp.full((LANES,), wv[4 * pp + j],
                                     dtype=jnp.float32)
                            for j in range(4)
                        ]
                        for s in range(SLICES):
                            sl = pl.ds(s * LANES, LANES)
                            a = (w[0] * buf[u, 4 * p + 0, sl]
                                 + w[1] * buf[u, 4 * p + 1, sl])
                            b2 = (w[2] * buf[u, 4 * p + 2, sl]
                                  + w[3] * buf[u, 4 * p + 3, sl])
                            outb[u, p, sl] = a + b2

                @pl.when(g + 2 < SUBS)
                def _():
                    start_gather(g + 2, u)

                pltpu.async_copy(
                    outb.at[u],
                    out_hbm.at[pl.ds(base + g * PTS_PER_SUB, PTS_PER_SUB)],
                    osems[u],
                )
            return carry

        lax.fori_loop(0, SUBS // 2, pair_body, 0, unroll=False)

        for u in range(2):
            pltpu.make_async_copy(
                outb.at[u],
                out_hbm.at[pl.ds(base + (SUBS - 2 + u) * PTS_PER_SUB,
                                 PTS_PER_SUB)],
                osems[u],
            ).wait()

    return k(table, idx3, wgt3)


def kernel(params, inds, wgts, freqs):
    table = params.reshape(NFREQS, NPIX).T
    idx3 = inds.astype(jnp.int32).reshape(NW, NIDX, ROWS_PER_G)
    wgt3 = wgts.astype(jnp.float32).reshape(NW, NIDX, ROWS_PER_G)
    out = _sc_gather(table, idx3, wgt3)
    return out.T.reshape(1, 1, 1, NFREQS, NPTS)


# speedup vs baseline: 3.0352x; 1.0004x over previous
import functools

import jax
import jax.numpy as jnp
from jax import lax
from jax.experimental import pallas as pl
from jax.experimental.pallas import tpu as pltpu
from jax.experimental.pallas import tpu_sc as plsc

NPIX = 196608
NFREQS = 128
NPTS = 65536

NW = 32
PTS_PER_W = NPTS // NW
PTS_PER_SUB = 64
GPS = 2
ROWS_PER_G = 128
SUBS = PTS_PER_W // PTS_PER_SUB
NIDX = SUBS * GPS
LANES = 16
SLICES = NFREQS // LANES


def _sc_gather(table, idx3, wgt3):
    mesh = plsc.VectorSubcoreMesh(core_axis_name="c", subcore_axis_name="s")

    @functools.partial(
        pl.kernel,
        out_type=jax.ShapeDtypeStruct((NPTS, NFREQS), jnp.float32),
        mesh=mesh,
        scratch_types=[
            pltpu.VMEM((NIDX * ROWS_PER_G,), jnp.int32),
            pltpu.VMEM((NIDX * ROWS_PER_G,), jnp.float32),
            pltpu.VMEM((2, PTS_PER_SUB * 4, NFREQS), jnp.float32),
            pltpu.VMEM((2, PTS_PER_SUB, NFREQS), jnp.float32),
            pltpu.SemaphoreType.DMA,
            pltpu.SemaphoreType.DMA,
            pltpu.SemaphoreType.DMA,
            pltpu.SemaphoreType.DMA,
        ],
    )
    def k(table_hbm, idx_hbm, wgt_hbm, out_hbm, idx_v, wgt_v, buf, outb,
          gsem0, gsem1, osem0, osem1):
        gsems = (gsem0, gsem1)
        osems = (osem0, osem1)
        wid = lax.axis_index("s") * 2 + lax.axis_index("c")
        base = wid * PTS_PER_W
        wbase = wid * (NIDX * ROWS_PER_G)
        pltpu.sync_copy(idx_hbm.at[pl.ds(wbase, NIDX * ROWS_PER_G)], idx_v)
        pltpu.sync_copy(wgt_hbm.at[pl.ds(wbase, NIDX * ROWS_PER_G)], wgt_v)

        def start_gather(g, u):
            for h in range(GPS):
                pltpu.async_copy(
                    table_hbm.at[idx_v.at[pl.ds((GPS * g + h) * ROWS_PER_G,
                                                ROWS_PER_G)]],
                    buf.at[u, pl.ds(h * ROWS_PER_G, ROWS_PER_G)],
                    gsems[u],
                )

        def wait_gather(g, u):
            for h in range(GPS):
                pltpu.make_async_copy(
                    table_hbm.at[idx_v.at[pl.ds((GPS * g + h) * ROWS_PER_G,
                                                ROWS_PER_G)]],
                    buf.at[u, pl.ds(h * ROWS_PER_G, ROWS_PER_G)],
                    gsems[u],
                ).wait()

        for u in range(2):
            start_gather(u, u)

        def pair_body(tq, carry):
            for u in range(2):
                g = 2 * tq + u
                wait_gather(g, u)

                @pl.when(tq >= 1)
                def _():
                    pltpu.make_async_copy(
                        outb.at[u],
                        out_hbm.at[pl.ds(base + (g - 2) * PTS_PER_SUB,
                                         PTS_PER_SUB)],
                        osems[u],
                    ).wait()

                @plsc.parallel_loop(0, PTS_PER_SUB // 4, unroll=2)
                def _(q, u=u):
                    wv = wgt_v[pl.ds((GPS * g + q // 8) * ROWS_PER_G
                                     + (q % 8) * LANES, LANES)]
                    for pp in range(4):
                        p = q * 4 + pp
                        w = [
                            jnp.full((LANES,), wv[4 * pp + j],
                                     dtype=jnp.float32)
                            for j in range(4)
                        ]
                        for s in range(SLICES):
                            sl = pl.ds(s * LANES, LANES)
                            a = (w[0] * buf[u, 4 * p + 0, sl]
                                 + w[1] * buf[u, 4 * p + 1, sl])
                            b2 = (w[2] * buf[u, 4 * p + 2, sl]
                                  + w[3] * buf[u, 4 * p + 3, sl])
                            outb[u, p, sl] = a + b2

                @pl.when(g + 2 < SUBS)
                def _():
                    start_gather(g + 2, u)

                pltpu.async_copy(
                    outb.at[u],
                    out_hbm.at[pl.ds(base + g * PTS_PER_SUB, PTS_PER_SUB)],
                    osems[u],
                )
            return carry

        lax.fori_loop(0, SUBS // 2, pair_body, 0, unroll=False)

        for u in range(2):
            pltpu.make_async_copy(
                outb.at[u],
                out_hbm.at[pl.ds(base + (SUBS - 2 + u) * PTS_PER_SUB,
                                 PTS_PER_SUB)],
                osems[u],
            ).wait()

    return k(table, idx3, wgt3)


def kernel(params, inds, wgts, freqs):
    table = params.reshape(NFREQS, NPIX).T
    idx3 = inds.astype(jnp.int32).reshape(-1)
    wgt3 = wgts.astype(jnp.float32).reshape(-1)
    out = _sc_gather(table, idx3, wgt3)
    return out.T.reshape(1, 1, 1, NFREQS, NPTS)
